# baseline (speedup 1.0000x reference)
import functools

import jax
import jax.numpy as jnp
from jax import lax
from jax.experimental import pallas as pl
from jax.experimental.pallas import tpu as pltpu
from jax.experimental.pallas import tpu_sc as plsc

D_MODEL = 64
N_BITS_PAD = 32
LUT = 1024
CHUNK = 512
SUB = 128


def _lut_body(x_ref, t_ref, o_ref):
    xrow = x_ref[0]
    iot = jax.lax.broadcasted_iota(jnp.int32, (N_BITS_PAD, 1), 0)
    bits_t = ((xrow >> iot) & 1).astype(jnp.float32)
    o_ref[...] = jax.lax.dot_general(
        bits_t,
        t_ref[...],
        (((0,), (0,)), ((), ())),
        preferred_element_type=jnp.float32,
    )


def _build_lut(table):
    u = jnp.arange(LUT, dtype=jnp.int32)
    x_lut = jnp.concatenate([u, u << 10]).reshape(1, 2 * LUT)
    tpad = jnp.zeros((N_BITS_PAD, D_MODEL), table.dtype)
    tpad = tpad.at[: table.shape[0]].set(table)
    return pl.pallas_call(
        _lut_body,
        grid=(1,),
        in_specs=[
            pl.BlockSpec((1, 2 * LUT), lambda i: (0, 0)),
            pl.BlockSpec((N_BITS_PAD, D_MODEL), lambda i: (0, 0)),
        ],
        out_specs=pl.BlockSpec((2 * LUT, D_MODEL), lambda i: (0, 0)),
        out_shape=jax.ShapeDtypeStruct((2 * LUT, D_MODEL), jnp.float32),
    )(x_lut, tpad)


def _sc_kernel(n):
    info = plsc.get_sparse_core_info()
    nw = info.num_cores * info.num_subcores
    per_w = n // nw
    n_pairs = per_w // (2 * CHUNK)
    mesh = plsc.VectorSubcoreMesh(core_axis_name="c", subcore_axis_name="s")

    @functools.partial(
        pl.kernel,
        mesh=mesh,
        out_type=jax.ShapeDtypeStruct((n, D_MODEL), jnp.float32),
        compiler_params=pltpu.CompilerParams(use_tc_tiling_on_sc=False),
        scratch_types=[
            pltpu.VMEM((2 * CHUNK,), jnp.int32),
            pltpu.VMEM((CHUNK,), jnp.int32),
            pltpu.VMEM((CHUNK,), jnp.int32),
            pltpu.VMEM((CHUNK,), jnp.int32),
            pltpu.VMEM((CHUNK,), jnp.int32),
            pltpu.VMEM((CHUNK, D_MODEL), jnp.float32),
            pltpu.VMEM((CHUNK, D_MODEL), jnp.float32),
            pltpu.SemaphoreType.DMA,
            pltpu.SemaphoreType.DMA,
            pltpu.SemaphoreType.DMA,
            pltpu.SemaphoreType.DMA,
        ],
    )
    def k(
        x_hbm, t_hbm, out_hbm,
        x_v, i0a_v, i1a_v, i0b_v, i1b_v, rows_a, rows_b,
        sem_a, sem_b, st_a, st_b,
    ):
        wid = lax.axis_index("s") * info.num_cores + lax.axis_index("c")
        w_base = wid * per_w

        def pair_body(p, _):
            base = w_base + p * 2 * CHUNK
            pltpu.sync_copy(x_hbm.at[pl.ds(base, 2 * CHUNK)], x_v)

            @plsc.parallel_loop(0, CHUNK // 16, 1, unroll=2)
            def idx_body(g):
                sl = pl.ds(g * 16, 16)
                v = x_v[sl]
                i0a_v[sl] = v & (LUT - 1)
                i1a_v[sl] = ((v >> 10) & (LUT - 1)) + LUT
                sl2 = pl.ds(CHUNK + g * 16, 16)
                w = x_v[sl2]
                i0b_v[sl] = w & (LUT - 1)
                i1b_v[sl] = ((w >> 10) & (LUT - 1)) + LUT

            def fire(t_ref, idx_ref, rows_ref, sem, add):
                return [pltpu.async_copy(t_ref.at[idx_ref], rows_ref, sem, add=add)]

            ga = fire(t_hbm, i0a_v, rows_a, sem_a, False)
            gb = fire(t_hbm, i0b_v, rows_b, sem_b, False)
            for h in ga:
                h.wait()
            aa = fire(t_hbm, i1a_v, rows_a, sem_a, True)
            for h in gb:
                h.wait()
            ab = fire(t_hbm, i1b_v, rows_b, sem_b, True)
            for h in aa:
                h.wait()
            sa = pltpu.async_copy(rows_a, out_hbm.at[pl.ds(base, CHUNK)], st_a)
            for h in ab:
                h.wait()
            sb = pltpu.async_copy(rows_b, out_hbm.at[pl.ds(base + CHUNK, CHUNK)], st_b)
            sa.wait()
            sb.wait()
            return 0

        lax.fori_loop(0, n_pairs, pair_body, 0)

    return k


TC_BLOCK = 16384


def _tc_body(x_ref, t_ref, o_ref):
    xrow = x_ref[0]
    iot = jax.lax.broadcasted_iota(jnp.int32, (N_BITS_PAD, 1), 0)
    bits_t = ((xrow >> iot) & 1).astype(jnp.float32)
    o_ref[0] = jax.lax.dot_general(
        bits_t,
        t_ref[...],
        (((0,), (0,)), ((), ())),
        preferred_element_type=jnp.float32,
    )


def _tc_kernel(x_flat, table):
    n = x_flat.size
    nb = n // TC_BLOCK
    xf = x_flat.reshape(nb, 1, TC_BLOCK)
    tpad = jnp.zeros((N_BITS_PAD, D_MODEL), table.dtype).at[: table.shape[0]].set(table)
    out = pl.pallas_call(
        _tc_body,
        grid=(nb,),
        in_specs=[
            pl.BlockSpec((1, 1, TC_BLOCK), lambda i: (i, 0, 0)),
            pl.BlockSpec((N_BITS_PAD, D_MODEL), lambda i: (0, 0)),
        ],
        out_specs=pl.BlockSpec((1, TC_BLOCK, D_MODEL), lambda i: (i, 0, 0)),
        out_shape=jax.ShapeDtypeStruct((nb, TC_BLOCK, D_MODEL), jnp.float32),
    )(xf, tpad)
    return out.reshape(n, D_MODEL)


N_SC = 196608


def kernel(x, table):
    x_shape = x.shape
    n = x.size
    xf = x.reshape(n)
    t = _build_lut(table)
    sc_out = _sc_kernel(N_SC)(xf[:N_SC], t)
    tc_out = _tc_kernel(xf[N_SC:], table)
    out = jnp.concatenate([sc_out, tc_out], axis=0)
    return out.reshape(*x_shape, D_MODEL)


# speedup vs baseline: 3.8389x; 3.8389x over previous
import jax
import jax.numpy as jnp
from jax.experimental import pallas as pl

D_MODEL = 64
N_BITS_PAD = 32
BLOCK = 16384


def _body(x_ref, t_ref, o_ref):
    xrow = x_ref[0]
    iot = jax.lax.broadcasted_iota(jnp.int32, (N_BITS_PAD, 1), 0)
    bits_t = ((xrow >> iot) & 1).astype(jnp.float32)
    o_ref[0] = jax.lax.dot_general(
        bits_t,
        t_ref[...],
        (((0,), (0,)), ((), ())),
        preferred_element_type=jnp.float32,
    )


def kernel(x, table):
    x_shape = x.shape
    n = x.size
    assert n % BLOCK == 0, n
    nb = n // BLOCK
    xf = x.reshape(nb, 1, BLOCK)
    tpad = jnp.zeros((N_BITS_PAD, D_MODEL), table.dtype).at[: table.shape[0]].set(table)
    out = pl.pallas_call(
        _body,
        grid=(nb,),
        in_specs=[
            pl.BlockSpec((1, 1, BLOCK), lambda i: (i, 0, 0)),
            pl.BlockSpec((N_BITS_PAD, D_MODEL), lambda i: (0, 0)),
        ],
        out_specs=pl.BlockSpec((1, BLOCK, D_MODEL), lambda i: (i, 0, 0)),
        out_shape=jax.ShapeDtypeStruct((nb, BLOCK, D_MODEL), jnp.float32),
    )(xf, tpad)
    return out.reshape(*x_shape, D_MODEL)
